# baseline (speedup 1.0000x reference)
import functools

import jax
import jax.numpy as jnp
from jax import lax
from jax.experimental import pallas as pl
from jax.experimental.pallas import tpu as pltpu
from jax.experimental.pallas import tpu_sc as plsc

_B, _N, _C = 4, 16384, 4
_G = 128
_KF = 32
_R = 0.25
_KUP = 512
_OV = 0.7
_NR = 128
_NL = 128
_GB = 32
_NEG = float("-inf")

_SC_CORES = 2
_SC_SUBCORES = 16
_NWK = _SC_CORES * _SC_SUBCORES
_TOT = _B * _G * _KF
_BPW = _TOT // _NWK
_CHUNK = 128


def _rmax(a):
    return jnp.max(jnp.max(a, axis=2, keepdims=True), axis=1, keepdims=True)


def _rmin(a):
    return jnp.min(jnp.min(a, axis=2, keepdims=True), axis=1, keepdims=True)


def _rsum(a):
    return jnp.sum(jnp.sum(a, axis=2, keepdims=True), axis=1, keepdims=True)


def _fps_nms_body(len_ref, x_ref, y_ref, z_ref, cen_ref, l1_ref):
    x = x_ref[...]
    y = y_ref[...]
    z = z_ref[...]
    ri = lax.broadcasted_iota(jnp.int32, (1, _NR, _NL), 1)
    ci = lax.broadcasted_iota(jnp.int32, (1, _NR, _NL), 2)
    iota = ri * _NL + ci
    bi = lax.broadcasted_iota(jnp.int32, (_B, 1, 1), 0)
    nvv = jnp.where(bi == 0, len_ref[0],
                    jnp.where(bi == 1, len_ref[1],
                              jnp.where(bi == 2, len_ref[2], len_ref[3])))
    valid = iota < nvv

    sel0 = (iota == 0) & (bi >= 0)
    x0 = _rsum(jnp.where(sel0, x, 0.0))
    y0 = _rsum(jnp.where(sel0, y, 0.0))
    z0 = _rsum(jnp.where(sel0, z, 0.0))
    dx = x - x0
    dy = y - y0
    dz = z - z0
    dist = jnp.where(valid, (dx * dx + dy * dy) + dz * dz, _NEG)

    lane_g = lax.broadcasted_iota(jnp.int32, (1, 1, _G), 2)
    sub_g = lax.broadcasted_iota(jnp.int32, (1, _G, 1), 1)
    zr = jnp.zeros((_B, 1, _G), jnp.float32)
    zc = jnp.zeros((_B, _G, 1), jnp.float32)
    cxr = jnp.where(lane_g == 0, x0, zr)
    cyr = jnp.where(lane_g == 0, y0, zr)
    czr = jnp.where(lane_g == 0, z0, zr)
    cxc = jnp.where(sub_g == 0, x0, zc)
    cyc = jnp.where(sub_g == 0, y0, zc)
    czc = jnp.where(sub_g == 0, z0, zc)

    def fps_step(i, carry):
        dist, cxr, cyr, czr, cxc, cyc, czc = carry
        m = _rmax(dist)
        idxv = _rmin(jnp.where(dist >= m, iota, _N))
        msk = iota == idxv
        xn = _rsum(jnp.where(msk, x, 0.0))
        yn = _rsum(jnp.where(msk, y, 0.0))
        zn = _rsum(jnp.where(msk, z, 0.0))
        dx = x - xn
        dy = y - yn
        dz = z - zn
        d2 = (dx * dx + dy * dy) + dz * dz
        dist = jnp.where(valid, jnp.minimum(dist, d2), _NEG)
        cxr = jnp.where(lane_g == i, xn, cxr)
        cyr = jnp.where(lane_g == i, yn, cyr)
        czr = jnp.where(lane_g == i, zn, czr)
        cxc = jnp.where(sub_g == i, xn, cxc)
        cyc = jnp.where(sub_g == i, yn, cyc)
        czc = jnp.where(sub_g == i, zn, czc)
        return dist, cxr, cyr, czr, cxc, cyc, czc

    carry = (dist, cxr, cyr, czr, cxc, cyc, czc)
    _, cxr, cyr, czr, cxc, cyc, czc = lax.fori_loop(1, _G, fps_step, carry)

    dcx = cxc - cxr
    dcy = cyc - cyr
    dcz = czc - czr
    dc = jnp.sqrt((dcx * dcx + dcy * dcy) + dcz * dcz)
    close = dc < (2.0 * _R * (1.0 - _OV))
    closef = jnp.where(close, 1.0, 0.0)
    retain = jnp.where(lane_g == 0, jnp.ones((_B, 1, _G), jnp.float32), 0.0)

    def nms_step(i, retain):
        rowi = jnp.max(jnp.where(sub_g == i, closef, 0.0), axis=1, keepdims=True)
        hit = (retain > 0.0) & (lane_g < i) & (rowi > 0.0)
        supp = _rmax(jnp.where(hit, 1.0, 0.0)) > 0.0
        return jnp.where(lane_g == i, jnp.where(supp, 0.0, 1.0), retain)

    retain = lax.fori_loop(1, _G, nms_step, retain)
    l1 = _rsum(retain).astype(jnp.int32)

    lane2 = lax.broadcasted_iota(jnp.int32, (1, _G, _NL), 2)
    cen = jnp.where(lane2 == 0, cxc,
                    jnp.where(lane2 == 1, cyc,
                              jnp.where(lane2 == 2, czc, jnp.zeros((_B, _G, _NL), jnp.float32))))
    cen_ref[...] = cen
    l1_ref[...] = jnp.broadcast_to(l1, (_B, 8, 128))


def _run_fps_nms(lengths, x, y, z):
    return pl.pallas_call(
        _fps_nms_body,
        grid=(1,),
        in_specs=[
            pl.BlockSpec(memory_space=pltpu.SMEM),
            pl.BlockSpec((_B, _NR, _NL), lambda b: (0, 0, 0)),
            pl.BlockSpec((_B, _NR, _NL), lambda b: (0, 0, 0)),
            pl.BlockSpec((_B, _NR, _NL), lambda b: (0, 0, 0)),
        ],
        out_specs=[
            pl.BlockSpec((_B, _G, _NL), lambda b: (0, 0, 0)),
            pl.BlockSpec((_B, 8, 128), lambda b: (0, 0, 0)),
        ],
        out_shape=[
            jax.ShapeDtypeStruct((_B, _G, _NL), jnp.float32),
            jax.ShapeDtypeStruct((_B, 8, 128), jnp.int32),
        ],
    )(lengths, x, y, z)


def _ballq_body(len_ref, l1_ref, pts_ref, cen_ref, tops_ref, sg_ref):
    b = pl.program_id(0)
    gb = pl.program_id(1)
    nv = len_ref[b]
    l1 = l1_ref[b]

    px = pts_ref[0, 0:1, :]
    py = pts_ref[0, 1:2, :]
    pz = pts_ref[0, 2:3, :]
    pe = pts_ref[0, 3:4, :]
    cx = cen_ref[0][:, 0:1]
    cy = cen_ref[0][:, 1:2]
    cz = cen_ref[0][:, 2:3]

    dx = cx - px
    dy = cy - py
    dz = cz - pz
    d2 = (dx * dx + dy * dy) + dz * dz

    iota_n = lax.broadcasted_iota(jnp.int32, (1, _N), 1)
    sub = lax.broadcasted_iota(jnp.int32, (_GB, 1), 0)
    grow = gb * _GB + sub
    within = (d2 <= _R * _R) & (iota_n < nv) & (grow < l1)

    wf = jnp.where(within, 1.0, 0.0)
    rN = lax.broadcasted_iota(jnp.int32, (_N, _NL), 0)
    cN = lax.broadcasted_iota(jnp.int32, (_N, _NL), 1)
    bounds = jnp.where(rN < (cN + 1) * _NL, 1.0, 0.0)
    pref = jnp.dot(wf, bounds, preferred_element_type=jnp.float32)
    lane1 = lax.broadcasted_iota(jnp.int32, (1, _NL), 1)
    fb = jnp.min(jnp.where(pref > float(_KUP), lane1, _NL),
                 axis=1, keepdims=True)
    lo0 = fb * _NL
    hi0 = lo0 + _NL

    def bs_step(_, lohi):
        lo, hi = lohi
        mid = (lo + hi) // 2
        cnt = jnp.sum(jnp.where(iota_n < mid, wf, 0.0),
                      axis=1, keepdims=True)
        ok = cnt <= float(_KUP)
        return jnp.where(ok, mid, lo), jnp.where(ok, hi, mid)

    lo, _ = lax.fori_loop(0, 7, bs_step, (lo0, hi0))
    keep = within & (iota_n < lo)

    key = jnp.where(keep, pe, _NEG)
    nonempty = jnp.max(jnp.where(keep, 1.0, 0.0), axis=1, keepdims=True) > 0.0
    lane = lax.broadcasted_iota(jnp.int32, (_GB, _NL), 1)

    def tk_step(j, carry):
        key, tops, idx0 = carry
        m = jnp.max(key, axis=1, keepdims=True)
        idxv = jnp.min(jnp.where(key >= m, iota_n, _N), axis=1, keepdims=True)
        validj = m > _NEG
        idx0 = jnp.where(j == 0, idxv, idx0)
        fill = jnp.where(validj, idxv, idx0)
        tops = jnp.where(lane == j, fill, tops)
        key = jnp.where(iota_n == idxv, _NEG, key)
        return key, tops, idx0

    tops0 = jnp.zeros((_GB, _NL), jnp.int32)
    idx00 = jnp.zeros((_GB, 1), jnp.int32)
    _, tops, _ = lax.fori_loop(0, _KF, tk_step, (key, tops0, idx00))

    tops = jnp.where(nonempty, tops, 0) + b * _N
    tops_ref[0] = tops
    sg_ref[0] = jnp.broadcast_to(jnp.where(nonempty, 1.0, 0.0), (_GB, _NL))


def _run_ballq(lengths, l1, ptsT, cen):
    return pl.pallas_call(
        _ballq_body,
        grid=(_B, _G // _GB),
        in_specs=[
            pl.BlockSpec(memory_space=pltpu.SMEM),
            pl.BlockSpec(memory_space=pltpu.SMEM),
            pl.BlockSpec((1, _C, _N), lambda b, gb: (b, 0, 0)),
            pl.BlockSpec((1, _GB, _NL), lambda b, gb: (b, gb, 0)),
        ],
        out_specs=[
            pl.BlockSpec((1, _GB, _NL), lambda b, gb: (b, gb, 0)),
            pl.BlockSpec((1, _GB, _NL), lambda b, gb: (b, gb, 0)),
        ],
        out_shape=[
            jax.ShapeDtypeStruct((_B, _G, _NL), jnp.int32),
            jax.ShapeDtypeStruct((_B, _G, _NL), jnp.float32),
        ],
    )(lengths, l1, ptsT, cen)


def _sc_gather_body(tab_ref, idx_ref, out_ref, idx_v, rows_v, sem):
    wid = lax.axis_index("s") * _SC_CORES + lax.axis_index("c")
    nrow = _BPW // _CHUNK
    base = wid * nrow
    pltpu.sync_copy(idx_ref.at[pl.ds(base, nrow)], idx_v)
    for j in range(nrow):
        pltpu.async_copy(
            tab_ref.at[idx_v.at[j]],
            rows_v.at[pl.ds(j * _CHUNK, _CHUNK)],
            sem,
        ).wait()
    pltpu.sync_copy(rows_v, out_ref.at[pl.ds(wid * _BPW, _BPW)])


def _sc_gather(table, idx2d):
    mesh = plsc.VectorSubcoreMesh(core_axis_name="c", subcore_axis_name="s")
    call = functools.partial(
        pl.kernel,
        mesh=mesh,
        compiler_params=pltpu.CompilerParams(use_tc_tiling_on_sc=False),
        out_type=jax.ShapeDtypeStruct((_TOT, _C), jnp.float32),
        scratch_types=[
            pltpu.VMEM((_BPW // _CHUNK, _CHUNK), jnp.int32),
            pltpu.VMEM((_BPW, _C), jnp.float32),
            pltpu.SemaphoreType.DMA,
        ],
    )(_sc_gather_body)
    return call(table, idx2d)


def _final_body(raw_ref, cen_ref, sg_ref, grp_ref, em_ref):
    raw = raw_ref[0]
    sg = sg_ref[0]
    cx = cen_ref[0][:, 0:1]
    cy = cen_ref[0][:, 1:2]
    cz = cen_ref[0][:, 2:3]
    lane = lax.broadcasted_iota(jnp.int32, (_G, _NL), 1)
    cm = lax.bitwise_and(lane, 3)
    off = jnp.where(cm == 0, cx,
                    jnp.where(cm == 1, cy,
                              jnp.where(cm == 2, cz, 0.0)))
    grp_ref[0] = (raw * sg - off) * 4.0

    gl = jnp.sum(sg[:, 0:1]).astype(jnp.int32)
    lane8 = lax.broadcasted_iota(jnp.int32, (8, 128), 1)
    em_ref[0] = jnp.where(lane8 < gl, 1, 0)


def _run_final(raw4, cen, sg):
    return pl.pallas_call(
        _final_body,
        grid=(_B,),
        in_specs=[
            pl.BlockSpec((1, _G, _NL), lambda b: (b, 0, 0)),
            pl.BlockSpec((1, _G, _NL), lambda b: (b, 0, 0)),
            pl.BlockSpec((1, _G, _NL), lambda b: (b, 0, 0)),
        ],
        out_specs=[
            pl.BlockSpec((1, _G, _NL), lambda b: (b, 0, 0)),
            pl.BlockSpec((1, 8, 128), lambda b: (b, 0, 0)),
        ],
        out_shape=[
            jax.ShapeDtypeStruct((_B, _G, _NL), jnp.float32),
            jax.ShapeDtypeStruct((_B, 8, 128), jnp.int32),
        ],
    )(raw4, cen, sg)


def kernel(points, lengths):
    x = points[:, :, 0].reshape(_B, _NR, _NL)
    y = points[:, :, 1].reshape(_B, _NR, _NL)
    z = points[:, :, 2].reshape(_B, _NR, _NL)
    ptsT = points.transpose(0, 2, 1)

    cen, l1b = _run_fps_nms(lengths, x, y, z)
    l1 = l1b[:, 0, 0]

    tops, sg = _run_ballq(lengths, l1, ptsT, cen)

    gidx = tops[:, :, :_KF].reshape(_TOT // _CHUNK, _CHUNK)
    table = points.reshape(_B * _N, _C)
    raw = _sc_gather(table, gidx)

    raw4 = raw.reshape(_B, _G, _NL)
    grp, em = _run_final(raw4, cen, sg)

    groups = grp.reshape(_B, _G, _KF, _C)
    centers = cen[:, :, :3]
    embedding_mask = em[:, 0, :] != 0
    return groups, centers, embedding_mask


# speedup vs baseline: 1.0033x; 1.0033x over previous
import functools

import jax
import jax.numpy as jnp
from jax import lax
from jax.experimental import pallas as pl
from jax.experimental.pallas import tpu as pltpu
from jax.experimental.pallas import tpu_sc as plsc

_B, _N, _C = 4, 16384, 4
_G = 128
_KF = 32
_R = 0.25
_KUP = 512
_OV = 0.7
_NR = 128
_NL = 128
_GB = 32
_NEG = float("-inf")

_SC_CORES = 2
_SC_SUBCORES = 16
_NWK = _SC_CORES * _SC_SUBCORES
_TOT = _B * _G * _KF
_BPW = _TOT // _NWK
_CHUNK = 128


def _rmax(a):
    return jnp.max(jnp.max(a, axis=2, keepdims=True), axis=1, keepdims=True)


def _rmin(a):
    return jnp.min(jnp.min(a, axis=2, keepdims=True), axis=1, keepdims=True)


def _rsum(a):
    return jnp.sum(jnp.sum(a, axis=2, keepdims=True), axis=1, keepdims=True)


def _fps_nms_body(len_ref, x_ref, y_ref, z_ref, cen_ref, l1_ref):
    x = x_ref[...]
    y = y_ref[...]
    z = z_ref[...]
    ri = lax.broadcasted_iota(jnp.int32, (1, _NR, _NL), 1)
    ci = lax.broadcasted_iota(jnp.int32, (1, _NR, _NL), 2)
    iota = ri * _NL + ci
    bi = lax.broadcasted_iota(jnp.int32, (_B, 1, 1), 0)
    nvv = jnp.where(bi == 0, len_ref[0],
                    jnp.where(bi == 1, len_ref[1],
                              jnp.where(bi == 2, len_ref[2], len_ref[3])))
    valid = iota < nvv

    sel0 = (iota == 0) & (bi >= 0)
    x0 = _rsum(jnp.where(sel0, x, 0.0))
    y0 = _rsum(jnp.where(sel0, y, 0.0))
    z0 = _rsum(jnp.where(sel0, z, 0.0))
    dx = x - x0
    dy = y - y0
    dz = z - z0
    dist = jnp.where(valid, (dx * dx + dy * dy) + dz * dz, _NEG)

    lane_g = lax.broadcasted_iota(jnp.int32, (1, 1, _G), 2)
    sub_g = lax.broadcasted_iota(jnp.int32, (1, _G, 1), 1)
    zr = jnp.zeros((_B, 1, _G), jnp.float32)
    zc = jnp.zeros((_B, _G, 1), jnp.float32)
    cxr = jnp.where(lane_g == 0, x0, zr)
    cyr = jnp.where(lane_g == 0, y0, zr)
    czr = jnp.where(lane_g == 0, z0, zr)
    cxc = jnp.where(sub_g == 0, x0, zc)
    cyc = jnp.where(sub_g == 0, y0, zc)
    czc = jnp.where(sub_g == 0, z0, zc)

    def fps_step(i, carry):
        dist, cxr, cyr, czr, cxc, cyc, czc = carry
        m = _rmax(dist)
        idxv = _rmin(jnp.where(dist >= m, iota, _N))
        msk = iota == idxv
        xn = _rsum(jnp.where(msk, x, 0.0))
        yn = _rsum(jnp.where(msk, y, 0.0))
        zn = _rsum(jnp.where(msk, z, 0.0))
        dx = x - xn
        dy = y - yn
        dz = z - zn
        d2 = (dx * dx + dy * dy) + dz * dz
        dist = jnp.where(valid, jnp.minimum(dist, d2), _NEG)
        cxr = jnp.where(lane_g == i, xn, cxr)
        cyr = jnp.where(lane_g == i, yn, cyr)
        czr = jnp.where(lane_g == i, zn, czr)
        cxc = jnp.where(sub_g == i, xn, cxc)
        cyc = jnp.where(sub_g == i, yn, cyc)
        czc = jnp.where(sub_g == i, zn, czc)
        return dist, cxr, cyr, czr, cxc, cyc, czc

    carry = (dist, cxr, cyr, czr, cxc, cyc, czc)
    _, cxr, cyr, czr, cxc, cyc, czc = lax.fori_loop(1, _G, fps_step, carry)

    dcx = cxc - cxr
    dcy = cyc - cyr
    dcz = czc - czr
    dc = jnp.sqrt((dcx * dcx + dcy * dcy) + dcz * dcz)
    close = dc < (2.0 * _R * (1.0 - _OV))
    closef = jnp.where(close, 1.0, 0.0)
    retain = jnp.where(lane_g == 0, jnp.ones((_B, 1, _G), jnp.float32), 0.0)

    def nms_step(i, retain):
        rowi = jnp.max(jnp.where(sub_g == i, closef, 0.0), axis=1, keepdims=True)
        hit = (retain > 0.0) & (lane_g < i) & (rowi > 0.0)
        supp = _rmax(jnp.where(hit, 1.0, 0.0)) > 0.0
        return jnp.where(lane_g == i, jnp.where(supp, 0.0, 1.0), retain)

    retain = lax.fori_loop(1, _G, nms_step, retain)
    l1 = _rsum(retain).astype(jnp.int32)

    lane2 = lax.broadcasted_iota(jnp.int32, (1, _G, _NL), 2)
    cen = jnp.where(lane2 == 0, cxc,
                    jnp.where(lane2 == 1, cyc,
                              jnp.where(lane2 == 2, czc, jnp.zeros((_B, _G, _NL), jnp.float32))))
    cen_ref[...] = cen
    l1_ref[...] = jnp.broadcast_to(l1, (_B, 8, 128))


def _run_fps_nms(lengths, x, y, z):
    return pl.pallas_call(
        _fps_nms_body,
        grid=(1,),
        in_specs=[
            pl.BlockSpec(memory_space=pltpu.SMEM),
            pl.BlockSpec((_B, _NR, _NL), lambda b: (0, 0, 0)),
            pl.BlockSpec((_B, _NR, _NL), lambda b: (0, 0, 0)),
            pl.BlockSpec((_B, _NR, _NL), lambda b: (0, 0, 0)),
        ],
        out_specs=[
            pl.BlockSpec((_B, _G, _NL), lambda b: (0, 0, 0)),
            pl.BlockSpec((_B, 8, 128), lambda b: (0, 0, 0)),
        ],
        out_shape=[
            jax.ShapeDtypeStruct((_B, _G, _NL), jnp.float32),
            jax.ShapeDtypeStruct((_B, 8, 128), jnp.int32),
        ],
    )(lengths, x, y, z)


def _ballq_body(len_ref, l1_ref, pts_ref, cen_ref, tops_ref, sg_ref):
    b = pl.program_id(0)
    gb = pl.program_id(1)
    nv = len_ref[b]
    l1 = l1_ref[b]

    px = pts_ref[0, 0:1, :]
    py = pts_ref[0, 1:2, :]
    pz = pts_ref[0, 2:3, :]
    pe = pts_ref[0, 3:4, :]
    cx = cen_ref[0][:, 0:1]
    cy = cen_ref[0][:, 1:2]
    cz = cen_ref[0][:, 2:3]

    dx = cx - px
    dy = cy - py
    dz = cz - pz
    d2 = (dx * dx + dy * dy) + dz * dz

    iota_n = lax.broadcasted_iota(jnp.int32, (1, _N), 1)
    sub = lax.broadcasted_iota(jnp.int32, (_GB, 1), 0)
    grow = gb * _GB + sub
    within = (d2 <= _R * _R) & (iota_n < nv) & (grow < l1)

    wf = jnp.where(within, 1.0, 0.0)
    rN = lax.broadcasted_iota(jnp.int32, (_N, _NL), 0)
    cN = lax.broadcasted_iota(jnp.int32, (_N, _NL), 1)
    bounds = jnp.where(rN < (cN + 1) * _NL, 1.0, 0.0)
    pref = jnp.dot(wf, bounds, preferred_element_type=jnp.float32)
    lane1 = lax.broadcasted_iota(jnp.int32, (1, _NL), 1)
    fb = jnp.min(jnp.where(pref > float(_KUP), lane1, _NL),
                 axis=1, keepdims=True)
    lo0 = fb * _NL
    hi0 = lo0 + _NL

    def bs_step(_, lohi):
        lo, hi = lohi
        mid = (lo + hi) // 2
        cnt = jnp.sum(jnp.where(iota_n < mid, wf, 0.0),
                      axis=1, keepdims=True)
        ok = cnt <= float(_KUP)
        return jnp.where(ok, mid, lo), jnp.where(ok, hi, mid)

    lo, _ = lax.fori_loop(0, 7, bs_step, (lo0, hi0))
    keep = within & (iota_n < lo)

    key = jnp.where(keep, pe, _NEG)
    nonempty = jnp.max(jnp.where(keep, 1.0, 0.0), axis=1, keepdims=True) > 0.0
    lane = lax.broadcasted_iota(jnp.int32, (_GB, _NL), 1)

    def tk_step(j, carry):
        key, tops, idx0 = carry
        m = jnp.max(key, axis=1, keepdims=True)
        idxv = jnp.min(jnp.where(key >= m, iota_n, _N), axis=1, keepdims=True)
        validj = m > _NEG
        idx0 = jnp.where(j == 0, idxv, idx0)
        fill = jnp.where(validj, idxv, idx0)
        tops = jnp.where(lane == j, fill, tops)
        key = jnp.where(iota_n == idxv, _NEG, key)
        return key, tops, idx0

    tops0 = jnp.zeros((_GB, _NL), jnp.int32)
    idx00 = jnp.zeros((_GB, 1), jnp.int32)
    _, tops, _ = lax.fori_loop(0, _KF, tk_step, (key, tops0, idx00))

    tops = jnp.where(nonempty, tops, 0) + b * _N
    tops_ref[0] = tops
    sg_ref[0] = jnp.broadcast_to(jnp.where(nonempty, 1.0, 0.0), (_GB, _NL))


def _run_ballq(lengths, l1, ptsT, cen):
    return pl.pallas_call(
        _ballq_body,
        grid=(_B, _G // _GB),
        in_specs=[
            pl.BlockSpec(memory_space=pltpu.SMEM),
            pl.BlockSpec(memory_space=pltpu.SMEM),
            pl.BlockSpec((1, _C, _N), lambda b, gb: (b, 0, 0)),
            pl.BlockSpec((1, _GB, _NL), lambda b, gb: (b, gb, 0)),
        ],
        out_specs=[
            pl.BlockSpec((1, _GB, _NL), lambda b, gb: (b, gb, 0)),
            pl.BlockSpec((1, _GB, _NL), lambda b, gb: (b, gb, 0)),
        ],
        out_shape=[
            jax.ShapeDtypeStruct((_B, _G, _NL), jnp.int32),
            jax.ShapeDtypeStruct((_B, _G, _NL), jnp.float32),
        ],
    )(lengths, l1, ptsT, cen)


def _sc_gather_body(tab_ref, idx_ref, out_ref, idx_v, rows_v, sem):
    wid = lax.axis_index("s") * _SC_CORES + lax.axis_index("c")
    nrow = _BPW // _CHUNK
    base = wid * nrow
    pltpu.sync_copy(idx_ref.at[pl.ds(base, nrow)], idx_v)
    for j in range(nrow):
        pltpu.async_copy(
            tab_ref.at[idx_v.at[j]],
            rows_v.at[pl.ds(j * _CHUNK, _CHUNK)],
            sem,
        ).wait()
    pltpu.sync_copy(rows_v, out_ref.at[pl.ds(wid * _BPW, _BPW)])


def _sc_gather(table, idx2d):
    mesh = plsc.VectorSubcoreMesh(core_axis_name="c", subcore_axis_name="s")
    call = functools.partial(
        pl.kernel,
        mesh=mesh,
        compiler_params=pltpu.CompilerParams(use_tc_tiling_on_sc=False),
        out_type=jax.ShapeDtypeStruct((_TOT, 16), jnp.float32),
        scratch_types=[
            pltpu.VMEM((_BPW // _CHUNK, _CHUNK), jnp.int32),
            pltpu.VMEM((_BPW, 16), jnp.float32),
            pltpu.SemaphoreType.DMA,
        ],
    )(_sc_gather_body)
    return call(table, idx2d)


def _final_body(raw_ref, cen_ref, sg_ref, grp_ref, em_ref):
    raw = raw_ref[0]
    sg = sg_ref[0]
    cx = cen_ref[0][:, 0:1]
    cy = cen_ref[0][:, 1:2]
    cz = cen_ref[0][:, 2:3]
    lane = lax.broadcasted_iota(jnp.int32, (_G, _NL), 1)
    cm = lax.bitwise_and(lane, 3)
    off = jnp.where(cm == 0, cx,
                    jnp.where(cm == 1, cy,
                              jnp.where(cm == 2, cz, 0.0)))
    grp_ref[0] = (raw * sg - off) * 4.0

    gl = jnp.sum(sg[:, 0:1]).astype(jnp.int32)
    lane8 = lax.broadcasted_iota(jnp.int32, (8, 128), 1)
    em_ref[0] = jnp.where(lane8 < gl, 1, 0)


def _run_final(raw4, cen, sg):
    return pl.pallas_call(
        _final_body,
        grid=(_B,),
        in_specs=[
            pl.BlockSpec((1, _G, _NL), lambda b: (b, 0, 0)),
            pl.BlockSpec((1, _G, _NL), lambda b: (b, 0, 0)),
            pl.BlockSpec((1, _G, _NL), lambda b: (b, 0, 0)),
        ],
        out_specs=[
            pl.BlockSpec((1, _G, _NL), lambda b: (b, 0, 0)),
            pl.BlockSpec((1, 8, 128), lambda b: (b, 0, 0)),
        ],
        out_shape=[
            jax.ShapeDtypeStruct((_B, _G, _NL), jnp.float32),
            jax.ShapeDtypeStruct((_B, 8, 128), jnp.int32),
        ],
    )(raw4, cen, sg)


def kernel(points, lengths):
    x = points[:, :, 0].reshape(_B, _NR, _NL)
    y = points[:, :, 1].reshape(_B, _NR, _NL)
    z = points[:, :, 2].reshape(_B, _NR, _NL)
    ptsT = points.transpose(0, 2, 1)

    cen, l1b = _run_fps_nms(lengths, x, y, z)
    l1 = l1b[:, 0, 0]

    tops, sg = _run_ballq(lengths, l1, ptsT, cen)

    gidx = tops[:, :, :_KF].reshape(_TOT // _CHUNK, _CHUNK)
    table = jnp.pad(points, ((0, 0), (0, 0), (0, 16 - _C))).reshape(_B * _N, 16)
    raw = _sc_gather(table, gidx)

    raw4 = raw[:, :_C].reshape(_B, _G, _NL)
    grp, em = _run_final(raw4, cen, sg)

    groups = grp.reshape(_B, _G, _KF, _C)
    centers = cen[:, :, :3]
    embedding_mask = em[:, 0, :] != 0
    return groups, centers, embedding_mask
